# Initial kernel scaffold; baseline (speedup 1.0000x reference)
import jax
import jax.numpy as jnp
from jax.experimental import pallas as pl


def kernel(x, memory0, W_in, b_in, Wc1, bc1, Wc2, bc2, W_out, b_out):
    raise NotImplementedError("write your pallas kernel here")


# speedup vs baseline: 9.0348x; 9.0348x over previous
import functools

import jax
import jax.numpy as jnp
from jax.experimental import pallas as pl
from jax.experimental.pallas import tpu as pltpu

_B, _S, _D, _H = 4, 2048, 1024, 16
_HD = _D // _H
_C = 64
_NC = _S // _C
_R = 256
_F32 = jnp.float32


def _proj_kernel(x_ref, win_ref, bin_ref, wc1_ref, bc1_ref, wc2_ref, bc2_ref,
                 base_ref, zc_ref):
    x = x_ref[...]
    h1 = jnp.dot(x, wc1_ref[...], preferred_element_type=_F32) + bc1_ref[...][None, :]
    h1 = h1 * jax.nn.sigmoid(h1)
    zc = jnp.dot(h1, wc2_ref[...], preferred_element_type=_F32) + bc2_ref[...][None, :]
    base = jnp.dot(x, win_ref[...], preferred_element_type=_F32) + bin_ref[...][None, :]
    base_ref[...] = base
    zc_ref[...] = zc


def _scan_kernel(kvq_ref, sc_ref, m0_ref, wout_ref, bout_ref,
                 out_ref, mf_ref, mst):
    j = pl.program_id(1)

    @pl.when(j == 0)
    def _():
        mst[...] = m0_ref[0]

    iota_t = jax.lax.broadcasted_iota(jnp.int32, (_C, _C), 0)
    iota_r = jax.lax.broadcasted_iota(jnp.int32, (_C, _C), 1)
    strict_lower = iota_r < iota_t
    row_idx = jax.lax.broadcasted_iota(jnp.int32, (_C, 1), 0)

    ys = []
    for h in range(_H):
        kb = kvq_ref[0, h, 0]
        vb = kvq_ref[0, h, 1]
        qb = kvq_ref[0, h, 2]
        scv = sc_ref[0, h]
        k = kb * scv[:, 0:1]
        v = vb * scv[:, 1:2]
        q = qb * scv[:, 2:3]
        eta = jax.nn.sigmoid(scv[:, 3:4])
        z4 = scv[:, 4:5]
        la = jnp.minimum(z4, 0.0) - jnp.log(1.0 + jnp.exp(-jnp.abs(z4)))

        cs = la
        for sh in (1, 2, 4, 8, 16, 32):
            rolled = pltpu.roll(cs, sh, axis=0)
            cs = cs + jnp.where(row_idx >= sh, rolled, 0.0)
        cs0 = cs - la
        p = jnp.exp(cs0)
        cs_row = jnp.swapaxes(cs, 0, 1)
        dmat = jnp.exp(jnp.where(strict_lower, cs0 - cs_row, -1e30))

        mh = mst[h]
        kq = jnp.concatenate([k, q], axis=0)
        kqk = jax.lax.dot_general(kq, k, (((1,), (1,)), ((), ())),
                                  preferred_element_type=_F32)
        kq0 = jax.lax.dot_general(kq, mh, (((1,), (1,)), ((), ())),
                                  preferred_element_type=_F32)
        kk = kqk[:_C]
        qk = kqk[_C:]
        c0 = kq0[:_C]
        q0 = kq0[_C:]

        nm = eta * dmat * kk
        x_sol = eta * (p * c0 - v)
        w = nm
        for i in range(6):
            x_sol = x_sol + jnp.dot(w, x_sol, preferred_element_type=_F32)
            if i < 5:
                w = jnp.dot(w, w, preferred_element_type=_F32)

        y = p * q0 + jnp.dot(dmat * qk, x_sol, preferred_element_type=_F32)
        ys.append(y)

        cst = jnp.min(cs, axis=0, keepdims=True)
        e = jnp.exp(cst - cs)
        mst[h] = jnp.exp(cst) * mh + jax.lax.dot_general(
            x_sol * e, k, (((0,), (0,)), ((), ())), preferred_element_type=_F32)

    yall = jnp.concatenate(ys, axis=1)
    out_ref[0] = jnp.dot(yall, wout_ref[...], preferred_element_type=_F32) \
        + bout_ref[...][None, :]

    @pl.when(j == _NC - 1)
    def _():
        mf_ref[0] = mst[...]


def kernel(x, memory0, W_in, b_in, Wc1, bc1, Wc2, bc2, W_out, b_out):
    b, s, d = x.shape
    h, hd = _H, _HD
    x2 = x.reshape(b * s, d)
    wc2p = Wc2.reshape(d, h, 5).transpose(0, 2, 1).reshape(d, 5 * h)
    bc2p = bc2.reshape(h, 5).transpose(1, 0).reshape(5 * h)

    n_tiles = (b * s) // _R
    base, zc = pl.pallas_call(
        _proj_kernel,
        grid=(n_tiles,),
        in_specs=[
            pl.BlockSpec((_R, d), lambda i: (i, 0)),
            pl.BlockSpec((d, 3 * d), lambda i: (0, 0)),
            pl.BlockSpec((3 * d,), lambda i: (0,)),
            pl.BlockSpec((d, d), lambda i: (0, 0)),
            pl.BlockSpec((d,), lambda i: (0,)),
            pl.BlockSpec((d, 5 * h), lambda i: (0, 0)),
            pl.BlockSpec((5 * h,), lambda i: (0,)),
        ],
        out_specs=[
            pl.BlockSpec((_R, 3 * d), lambda i: (i, 0)),
            pl.BlockSpec((_R, 5 * h), lambda i: (i, 0)),
        ],
        out_shape=[
            jax.ShapeDtypeStruct((b * s, 3 * d), _F32),
            jax.ShapeDtypeStruct((b * s, 5 * h), _F32),
        ],
        compiler_params=pltpu.CompilerParams(
            dimension_semantics=("parallel",),
            vmem_limit_bytes=56 * 1024 * 1024,
        ),
        name="srt_proj",
    )(x2, W_in, b_in, Wc1, bc1, wc2p, bc2p)

    kvq = base.reshape(b, s, h, 3, hd).transpose(0, 2, 3, 1, 4)
    sc = zc.reshape(b, s, 5, h).transpose(0, 3, 1, 2)

    out, mf = pl.pallas_call(
        _scan_kernel,
        grid=(b, _NC),
        in_specs=[
            pl.BlockSpec((1, h, 3, _C, hd), lambda bi, j: (bi, 0, 0, j, 0)),
            pl.BlockSpec((1, h, _C, 5), lambda bi, j: (bi, 0, j, 0)),
            pl.BlockSpec((1, h, hd, hd), lambda bi, j: (bi, 0, 0, 0)),
            pl.BlockSpec((d, d), lambda bi, j: (0, 0)),
            pl.BlockSpec((d,), lambda bi, j: (0,)),
        ],
        out_specs=[
            pl.BlockSpec((1, _C, d), lambda bi, j: (bi, j, 0)),
            pl.BlockSpec((1, h, hd, hd), lambda bi, j: (bi, 0, 0, 0)),
        ],
        out_shape=[
            jax.ShapeDtypeStruct((b, s, d), _F32),
            jax.ShapeDtypeStruct((b, h, hd, hd), _F32),
        ],
        scratch_shapes=[pltpu.VMEM((h, hd, hd), _F32)],
        compiler_params=pltpu.CompilerParams(
            dimension_semantics=("parallel", "arbitrary"),
        ),
        name="srt_scan",
    )(kvq, sc, memory0, W_out, b_out)

    return out, mf


# speedup vs baseline: 10.3553x; 1.1462x over previous
import jax
import jax.numpy as jnp
from jax.experimental import pallas as pl
from jax.experimental.pallas import tpu as pltpu

_B, _S, _D, _H = 4, 2048, 1024, 16
_HD = _D // _H
_C = 64
_NC = _S // _C
_R = 256
_F32 = jnp.float32


def _proj_kernel(x_ref, win_ref, bin_ref, wc1_ref, bc1_ref, wc2_ref, bc2_ref,
                 e48_ref, kvq_ref, gate_ref):
    x = x_ref[...]
    h1 = jnp.dot(x, wc1_ref[...], preferred_element_type=_F32) + bc1_ref[...][None, :]
    h1 = h1 * jax.nn.sigmoid(h1)
    zc = jnp.dot(h1, wc2_ref[...], preferred_element_type=_F32) + bc2_ref[...][None, :]
    base = jnp.dot(x, win_ref[...], preferred_element_type=_F32) + bin_ref[...][None, :]
    scale = jnp.dot(zc[:, :48], e48_ref[...], preferred_element_type=_F32)
    kvq_ref[...] = base * scale
    gate_ref[...] = zc[:, 48:]


def _scan_kernel(kvq_ref, g_ref, m0_ref, wout_ref, bout_ref,
                 out_ref, mf_ref, mst, yb):
    j = pl.program_id(1)

    @pl.when(j == 0)
    def _():
        mst[...] = m0_ref[0]

    iota_t = jax.lax.broadcasted_iota(jnp.int32, (_C, _C), 0)
    iota_r = jax.lax.broadcasted_iota(jnp.int32, (_C, _C), 1)
    strict_lower = iota_r < iota_t

    z3 = g_ref[0, 0]
    z4 = g_ref[0, 1]
    eta_all = jax.nn.sigmoid(z3)
    la = jnp.minimum(z4, 0.0) - jnp.log(1.0 + jnp.exp(-jnp.abs(z4)))
    row_idx = jax.lax.broadcasted_iota(jnp.int32, (_C, 1), 0)
    cs = la
    for sh in (1, 2, 4, 8, 16, 32):
        rolled = pltpu.roll(cs, sh, axis=0)
        cs = cs + jnp.where(row_idx >= sh, rolled, 0.0)
    p_all = jnp.exp(cs - la)
    cst = jnp.min(cs, axis=0, keepdims=True)
    e_all = jnp.exp(cst - cs)
    ac_all = jnp.exp(cst)
    einv_t = jnp.exp(jnp.minimum(-jnp.swapaxes(cs, 0, 1), 80.0))

    for h in range(_H):
        k = kvq_ref[0, h, 0]
        v = kvq_ref[0, h, 1]
        q = kvq_ref[0, h, 2]
        p = p_all[:, h:h + 1]
        eta = eta_all[:, h:h + 1]
        e = e_all[:, h:h + 1]
        einv_r = einv_t[h:h + 1, :]

        kp = k * p
        qp = q * p
        kqp = jnp.concatenate([kp, qp], axis=0)
        gram = jax.lax.dot_general(kqp, k, (((1,), (1,)), ((), ())),
                                   preferred_element_type=_F32) * einv_r
        mt = mst[h]
        kq0 = jnp.dot(kqp, mt, preferred_element_type=_F32)

        nm = jnp.where(strict_lower, gram[:_C] * eta, 0.0)
        ymat = jnp.where(strict_lower, gram[_C:], 0.0)
        x_sol = eta * (kq0[:_C] - v)
        w = nm
        for i in range(6):
            x_sol = x_sol + jnp.dot(w, x_sol, preferred_element_type=_F32)
            if i < 5:
                w = jnp.dot(w, w, preferred_element_type=_F32)

        yb[:, h * _HD:(h + 1) * _HD] = kq0[_C:] + jnp.dot(
            ymat, x_sol, preferred_element_type=_F32)

        mst[h] = ac_all[0:1, h:h + 1] * mt + jax.lax.dot_general(
            k, x_sol * e, (((0,), (0,)), ((), ())), preferred_element_type=_F32)

    out_ref[0] = jnp.dot(yb[...], wout_ref[...], preferred_element_type=_F32) \
        + bout_ref[...][None, :]

    @pl.when(j == _NC - 1)
    def _():
        mf_ref[0] = mst[...]


def kernel(x, memory0, W_in, b_in, Wc1, bc1, Wc2, bc2, W_out, b_out):
    b, s, d = x.shape
    h, hd = _H, _HD
    x2 = x.reshape(b * s, d)
    wc2r = Wc2.reshape(d, h, 5)
    wc2p = jnp.concatenate([wc2r[:, :, :3].reshape(d, 48),
                            wc2r[:, :, 3:].transpose(0, 2, 1).reshape(d, 32)], axis=1)
    bc2r = bc2.reshape(h, 5)
    bc2p = jnp.concatenate([bc2r[:, :3].reshape(48),
                            bc2r[:, 3:].transpose(1, 0).reshape(32)])
    e48 = jnp.repeat(jnp.eye(48, dtype=_F32), hd, axis=1)

    n_tiles = (b * s) // _R
    kvqs, gates = pl.pallas_call(
        _proj_kernel,
        grid=(n_tiles,),
        in_specs=[
            pl.BlockSpec((_R, d), lambda i: (i, 0)),
            pl.BlockSpec((d, 3 * d), lambda i: (0, 0)),
            pl.BlockSpec((3 * d,), lambda i: (0,)),
            pl.BlockSpec((d, d), lambda i: (0, 0)),
            pl.BlockSpec((d,), lambda i: (0,)),
            pl.BlockSpec((d, 80), lambda i: (0, 0)),
            pl.BlockSpec((80,), lambda i: (0,)),
            pl.BlockSpec((48, 3 * d), lambda i: (0, 0)),
        ],
        out_specs=[
            pl.BlockSpec((_R, 3 * d), lambda i: (i, 0)),
            pl.BlockSpec((_R, 32), lambda i: (i, 0)),
        ],
        out_shape=[
            jax.ShapeDtypeStruct((b * s, 3 * d), _F32),
            jax.ShapeDtypeStruct((b * s, 32), _F32),
        ],
        compiler_params=pltpu.CompilerParams(
            dimension_semantics=("parallel",),
            vmem_limit_bytes=56 * 1024 * 1024,
        ),
        name="srt_proj",
    )(x2, W_in, b_in, Wc1, bc1, wc2p, bc2p, e48)

    kvq = kvqs.reshape(b, s, h, 3, hd).transpose(0, 2, 3, 1, 4)
    sc = gates.reshape(b, s, 2, h).transpose(0, 2, 1, 3)
    m0t = memory0.transpose(0, 1, 3, 2)

    out, mft = pl.pallas_call(
        _scan_kernel,
        grid=(b, _NC),
        in_specs=[
            pl.BlockSpec((1, h, 3, _C, hd), lambda bi, j: (bi, 0, 0, j, 0)),
            pl.BlockSpec((1, 2, _C, h), lambda bi, j: (bi, 0, j, 0)),
            pl.BlockSpec((1, h, hd, hd), lambda bi, j: (bi, 0, 0, 0)),
            pl.BlockSpec((d, d), lambda bi, j: (0, 0)),
            pl.BlockSpec((d,), lambda bi, j: (0,)),
        ],
        out_specs=[
            pl.BlockSpec((1, _C, d), lambda bi, j: (bi, j, 0)),
            pl.BlockSpec((1, h, hd, hd), lambda bi, j: (bi, 0, 0, 0)),
        ],
        out_shape=[
            jax.ShapeDtypeStruct((b, s, d), _F32),
            jax.ShapeDtypeStruct((b, h, hd, hd), _F32),
        ],
        scratch_shapes=[pltpu.VMEM((h, hd, hd), _F32),
                        pltpu.VMEM((_C, d), _F32)],
        compiler_params=pltpu.CompilerParams(
            dimension_semantics=("parallel", "arbitrary"),
        ),
        name="srt_scan",
    )(kvq, sc, m0t, W_out, b_out)

    return out, mft.transpose(0, 1, 3, 2)


# speedup vs baseline: 30.4875x; 2.9442x over previous
import jax
import jax.numpy as jnp
from jax.experimental import pallas as pl
from jax.experimental.pallas import tpu as pltpu

_B, _S, _D, _H = 4, 2048, 1024, 16
_HD = _D // _H
_C = 64
_NC = _S // _C
_R = 256
_F32 = jnp.float32


def _proj_kernel(x_ref, win_ref, bin_ref, wc1_ref, bc1_ref, wc2_ref, bc2_ref,
                 e48_ref, kvq_ref, gate_ref):
    x = x_ref[...]
    h1 = jnp.dot(x, wc1_ref[...], preferred_element_type=_F32) + bc1_ref[...][None, :]
    h1 = h1 * jax.nn.sigmoid(h1)
    zc = jnp.dot(h1, wc2_ref[...], preferred_element_type=_F32) + bc2_ref[...][None, :]
    base = jnp.dot(x, win_ref[...], preferred_element_type=_F32) + bin_ref[...][None, :]
    scale = jnp.dot(zc[:, :48], e48_ref[...], preferred_element_type=_F32)
    kvq_ref[...] = base * scale
    gate_ref[...] = zc[:, 48:]


def _scan_kernel(kvq_ref, g_ref, m0_ref, wout_ref, bout_ref,
                 out_ref, mf_ref, mst, yb):
    j = pl.program_id(1)

    @pl.when(j == 0)
    def _():
        mst[...] = m0_ref[0]

    iota_t = jax.lax.broadcasted_iota(jnp.int32, (_C, _C), 0)
    iota_r = jax.lax.broadcasted_iota(jnp.int32, (_C, _C), 1)
    strict_lower = iota_r < iota_t

    z3 = g_ref[0, 0]
    z4 = g_ref[0, 1]
    eta_all = jax.nn.sigmoid(z3)
    la = jnp.minimum(z4, 0.0) - jnp.log(1.0 + jnp.exp(-jnp.abs(z4)))
    row_idx = jax.lax.broadcasted_iota(jnp.int32, (_C, 1), 0)
    cs = la
    for sh in (1, 2, 4, 8, 16, 32):
        rolled = pltpu.roll(cs, sh, axis=0)
        cs = cs + jnp.where(row_idx >= sh, rolled, 0.0)
    p_all = jnp.exp(cs - la)
    cst = jnp.min(cs, axis=0, keepdims=True)
    e_all = jnp.exp(cst - cs)
    ac_all = jnp.exp(cst)
    einv_t = jnp.exp(jnp.minimum(-jnp.swapaxes(cs, 0, 1), 80.0))

    grp = 8
    for h0 in range(0, _H, grp):
        hs = range(h0, h0 + grp)
        ks, ymats, kq0s, xs, ws = {}, {}, {}, {}, {}
        for h in hs:
            k = kvq_ref[0, h, 0]
            v = kvq_ref[0, h, 1]
            q = kvq_ref[0, h, 2]
            p = p_all[:, h:h + 1]
            eta = eta_all[:, h:h + 1]
            einv_r = einv_t[h:h + 1, :]
            kp = k * p
            qp = q * p
            kqp = jnp.concatenate([kp, qp], axis=0)
            gram = jax.lax.dot_general(kqp, k, (((1,), (1,)), ((), ())),
                                       preferred_element_type=_F32) * einv_r
            kq0 = jnp.dot(kqp, mst[h], preferred_element_type=_F32)
            ks[h] = k
            ymats[h] = jnp.where(strict_lower, gram[_C:], 0.0)
            kq0s[h] = kq0
            xs[h] = eta * (kq0[:_C] - v)
            ws[h] = jnp.where(strict_lower, gram[:_C] * eta, 0.0)
        for i in range(6):
            for h in hs:
                xs[h] = xs[h] + jnp.dot(ws[h], xs[h], preferred_element_type=_F32)
                if i < 5:
                    ws[h] = jnp.dot(ws[h], ws[h], preferred_element_type=_F32)
        for h in hs:
            yb[:, h * _HD:(h + 1) * _HD] = kq0s[h][_C:] + jnp.dot(
                ymats[h], xs[h], preferred_element_type=_F32)
            mst[h] = ac_all[0:1, h:h + 1] * mst[h] + jax.lax.dot_general(
                ks[h], xs[h] * e_all[:, h:h + 1], (((0,), (0,)), ((), ())),
                preferred_element_type=_F32)

    out_ref[0] = jnp.dot(yb[...], wout_ref[...], preferred_element_type=_F32) \
        + bout_ref[...][None, :]

    @pl.when(j == _NC - 1)
    def _():
        mf_ref[0] = mst[...]


def kernel(x, memory0, W_in, b_in, Wc1, bc1, Wc2, bc2, W_out, b_out):
    b, s, d = x.shape
    h, hd = _H, _HD
    x2 = x.reshape(b * s, d)
    wc2r = Wc2.reshape(d, h, 5)
    wc2p = jnp.concatenate([wc2r[:, :, :3].reshape(d, 48),
                            wc2r[:, :, 3:].transpose(0, 2, 1).reshape(d, 32)], axis=1)
    bc2r = bc2.reshape(h, 5)
    bc2p = jnp.concatenate([bc2r[:, :3].reshape(48),
                            bc2r[:, 3:].transpose(1, 0).reshape(32)])
    e48 = jnp.repeat(jnp.eye(48, dtype=_F32), hd, axis=1)

    n_tiles = (b * s) // _R
    kvqs, gates = pl.pallas_call(
        _proj_kernel,
        grid=(n_tiles,),
        in_specs=[
            pl.BlockSpec((_R, d), lambda i: (i, 0)),
            pl.BlockSpec((d, 3 * d), lambda i: (0, 0)),
            pl.BlockSpec((3 * d,), lambda i: (0,)),
            pl.BlockSpec((d, d), lambda i: (0, 0)),
            pl.BlockSpec((d,), lambda i: (0,)),
            pl.BlockSpec((d, 80), lambda i: (0, 0)),
            pl.BlockSpec((80,), lambda i: (0,)),
            pl.BlockSpec((48, 3 * d), lambda i: (0, 0)),
        ],
        out_specs=[
            pl.BlockSpec((_R, 3 * d), lambda i: (i, 0)),
            pl.BlockSpec((_R, 32), lambda i: (i, 0)),
        ],
        out_shape=[
            jax.ShapeDtypeStruct((b * s, 3 * d), _F32),
            jax.ShapeDtypeStruct((b * s, 32), _F32),
        ],
        compiler_params=pltpu.CompilerParams(
            dimension_semantics=("parallel",),
            vmem_limit_bytes=56 * 1024 * 1024,
        ),
        name="srt_proj",
    )(x2, W_in, b_in, Wc1, bc1, wc2p, bc2p, e48)

    kvq = kvqs.reshape(b, s, h, 3, hd).transpose(0, 2, 3, 1, 4)
    sc = gates.reshape(b, s, 2, h).transpose(0, 2, 1, 3)
    m0t = memory0.transpose(0, 1, 3, 2)

    out, mft = pl.pallas_call(
        _scan_kernel,
        grid=(b, _NC),
        in_specs=[
            pl.BlockSpec((1, h, 3, _C, hd), lambda bi, j: (bi, 0, 0, j, 0)),
            pl.BlockSpec((1, 2, _C, h), lambda bi, j: (bi, 0, j, 0)),
            pl.BlockSpec((1, h, hd, hd), lambda bi, j: (bi, 0, 0, 0)),
            pl.BlockSpec((d, d), lambda bi, j: (0, 0)),
            pl.BlockSpec((d,), lambda bi, j: (0,)),
        ],
        out_specs=[
            pl.BlockSpec((1, _C, d), lambda bi, j: (bi, j, 0)),
            pl.BlockSpec((1, h, hd, hd), lambda bi, j: (bi, 0, 0, 0)),
        ],
        out_shape=[
            jax.ShapeDtypeStruct((b, s, d), _F32),
            jax.ShapeDtypeStruct((b, h, hd, hd), _F32),
        ],
        scratch_shapes=[pltpu.VMEM((h, hd, hd), _F32),
                        pltpu.VMEM((_C, d), _F32)],
        compiler_params=pltpu.CompilerParams(
            dimension_semantics=("parallel", "arbitrary"),
        ),
        name="srt_scan",
    )(kvq, sc, m0t, W_out, b_out)

    return out, mft.transpose(0, 1, 3, 2)
